# baseline (speedup 1.0000x reference)
import jax
import jax.numpy as jnp
from jax.experimental import pallas as pl

N = 10000
D = 128
BM = 200


def _gcn_block_kernel(x_ref, a1_ref, a2_ref, out_ref):
    xb = x_ref[...]
    out_ref[:, :D] = jnp.dot(a1_ref[...], xb, preferred_element_type=jnp.float32)
    out_ref[:, D:] = jnp.dot(a2_ref[...], xb, preferred_element_type=jnp.float32)


def kernel(x, adj_t, adj_t2):
    n, d = x.shape
    bm = BM if n % BM == 0 else n
    return pl.pallas_call(
        _gcn_block_kernel,
        grid=(n // bm,),
        in_specs=[
            pl.BlockSpec((n, d), lambda i: (0, 0)),
            pl.BlockSpec((bm, n), lambda i: (i, 0)),
            pl.BlockSpec((bm, n), lambda i: (i, 0)),
        ],
        out_specs=pl.BlockSpec((bm, 2 * d), lambda i: (i, 0)),
        out_shape=jax.ShapeDtypeStruct((n, 2 * d), jnp.float32),
    )(x, adj_t, adj_t2)
